# baseline (speedup 1.0000x reference)
import functools

import jax
import jax.numpy as jnp
from jax import lax
from jax.experimental import pallas as pl
from jax.experimental.pallas import tpu as pltpu
from jax.experimental.pallas import tpu_sc as plsc

N_NODES = 10000
N_EDGES = 320000
N_REL = 16
N_BASES = 4
DIM = 128

NC = 2
NS = 16
NW = NC * NS
K = 128
NCHT = N_EDGES // K
NCH = NCHT // NW
NCH_REM = NCHT % NW
ROWS_PER_TILE = 624


def _v_body(bw_ref, bases_ref, x_ref, src_ref, et_ref, v_ref, gidx_ref):
    r = pl.program_id(0)
    rows = lax.broadcasted_iota(jnp.int32, (N_REL, N_BASES), 0)
    bvec = jnp.sum(jnp.where(rows == r, bw_ref[...], 0.0), axis=0)
    w = bvec[0] * bases_ref[0]
    for b in range(1, N_BASES):
        w = w + bvec[b] * bases_ref[b]
    v_ref[0] = jnp.dot(x_ref[...], w.astype(jnp.bfloat16),
                       preferred_element_type=jnp.float32)

    @pl.when(r == 0)
    def _():
        gidx_ref[...] = et_ref[...] * N_NODES + src_ref[...]


def _compute_v_gidx(x, base_weights, bases, source, edge_type):
    src2 = source.reshape(N_EDGES // 128, 128)
    et2 = edge_type.reshape(N_EDGES // 128, 128)
    return pl.pallas_call(
        _v_body,
        grid=(N_REL,),
        in_specs=[
            pl.BlockSpec((N_REL, N_BASES), lambda r: (0, 0)),
            pl.BlockSpec((N_BASES, DIM, DIM), lambda r: (0, 0, 0)),
            pl.BlockSpec((N_NODES, DIM), lambda r: (0, 0)),
            pl.BlockSpec((N_EDGES // 128, 128), lambda r: (0, 0)),
            pl.BlockSpec((N_EDGES // 128, 128), lambda r: (0, 0)),
        ],
        out_specs=[
            pl.BlockSpec((1, N_NODES, DIM), lambda r: (r, 0, 0)),
            pl.BlockSpec((N_EDGES // 128, 128), lambda r: (0, 0)),
        ],
        out_shape=[
            jax.ShapeDtypeStruct((N_REL, N_NODES, DIM), jnp.float32),
            jax.ShapeDtypeStruct((N_EDGES // 128, 128), jnp.int32),
        ],
    )(base_weights, bases, x, src2, et2)


def _sc_edge_body(gidx_hbm, tgt_hbm, v_hbm, out_hbm,
                  gidx_v, tslot0, tslot1, rows0, rows1,
                  acc, sem0, sem1, semt0, semt1, semi):
    c = lax.axis_index("c")
    s = lax.axis_index("s")
    wid = c * NS + s
    n = NCH + jnp.where(wid < NCH_REM, 1, 0)

    def cid(i):
        return jnp.where(i < NCH, wid * NCH + i, NW * NCH + wid)

    idx_cp = pltpu.async_copy(gidx_hbm.at[pl.ds(wid * NCH * K, NCH * K)],
                              gidx_v.at[pl.ds(0, NCH * K)], semi)
    t_cp0 = pltpu.async_copy(tgt_hbm.at[wid * NCH + 0], tslot0, semt0)
    t_cp1 = pltpu.async_copy(tgt_hbm.at[wid * NCH + 1], tslot1, semt1)

    @pl.when(wid < NCH_REM)
    def _():
        pltpu.async_copy(gidx_hbm.at[pl.ds((NW * NCH + wid) * K, K)],
                         gidx_v.at[pl.ds(NCH * K, K)], semi)

    zero = jnp.zeros((16,), jnp.float32)
    for i in range(K):
        for j in range(DIM // 16):
            rows0[i, pl.ds(j * 16, 16)] = zero

    row0 = s * ROWS_PER_TILE
    for blk in range(4):
        pltpu.sync_copy(rows0, acc.at[pl.ds(row0 + blk * K, K), :])
    pltpu.sync_copy(rows0.at[pl.ds(0, ROWS_PER_TILE - 4 * K), :],
                    acc.at[pl.ds(row0 + 4 * K, ROWS_PER_TILE - 4 * K), :])

    @pl.when(s == 0)
    def _():
        pltpu.sync_copy(rows0.at[pl.ds(0, N_NODES - NS * ROWS_PER_TILE), :],
                        acc.at[pl.ds(NS * ROWS_PER_TILE,
                                     N_NODES - NS * ROWS_PER_TILE), :])

    idx_cp.wait()

    @pl.when(wid < NCH_REM)
    def _():
        pltpu.make_async_copy(gidx_hbm.at[pl.ds(0, K)],
                              gidx_v.at[pl.ds(NCH * K, K)], semi).wait()

    plsc.subcore_barrier()

    def fire_gather(i, rows, sem):
        pltpu.async_copy(v_hbm.at[gidx_v.at[pl.ds(i * K, K)]], rows, sem)

    fire_gather(0, rows0, sem0)
    fire_gather(1, rows1, sem1)

    def step(i, rows, tslot, sem, semt):
        pltpu.make_async_copy(v_hbm.at[gidx_v.at[pl.ds(0, K)]], rows, sem).wait()
        pltpu.make_async_copy(tgt_hbm.at[0], tslot, semt).wait()
        pltpu.sync_copy(rows, acc.at[tslot.at[0]], add=True)

        @pl.when(i + 2 < n)
        def _():
            pltpu.async_copy(tgt_hbm.at[cid(i + 2)], tslot, semt)
            fire_gather(i + 2, rows, sem)

    def pair(j, _):
        step(2 * j, rows0, tslot0, sem0, semt0)
        step(2 * j + 1, rows1, tslot1, sem1, semt1)
        return 0

    lax.fori_loop(0, NCH // 2, pair, 0)

    @pl.when(n > NCH)
    def _():
        pltpu.make_async_copy(v_hbm.at[gidx_v.at[pl.ds(0, K)]], rows0, sem0).wait()
        pltpu.make_async_copy(tgt_hbm.at[0], tslot0, semt0).wait()
        pltpu.sync_copy(rows0, acc.at[tslot0.at[0]], add=True)

    plsc.subcore_barrier()

    pltpu.sync_copy(acc.at[pl.ds(row0, ROWS_PER_TILE), :],
                    out_hbm.at[c, pl.ds(row0, ROWS_PER_TILE), :])

    @pl.when(s == 0)
    def _():
        pltpu.sync_copy(acc.at[pl.ds(NS * ROWS_PER_TILE, N_NODES - NS * ROWS_PER_TILE), :],
                        out_hbm.at[c, pl.ds(NS * ROWS_PER_TILE,
                                            N_NODES - NS * ROWS_PER_TILE), :])


def _sc_edge(gidx, target, v):
    mesh = plsc.VectorSubcoreMesh(core_axis_name="c", subcore_axis_name="s")
    kern = pl.kernel(
        _sc_edge_body,
        out_type=jax.ShapeDtypeStruct((NC, N_NODES, DIM), jnp.float32),
        mesh=mesh,
        scratch_types=[
            pltpu.VMEM(((NCH + 1) * K,), jnp.int32),
            pltpu.VMEM((1, K), jnp.int32),
            pltpu.VMEM((1, K), jnp.int32),
            pltpu.VMEM((K, DIM), jnp.float32),
            pltpu.VMEM((K, DIM), jnp.float32),
            pltpu.VMEM_SHARED((N_NODES, DIM), jnp.float32),
            pltpu.SemaphoreType.DMA,
            pltpu.SemaphoreType.DMA,
            pltpu.SemaphoreType.DMA,
            pltpu.SemaphoreType.DMA,
            pltpu.SemaphoreType.DMA,
        ],
    )
    t3 = target.reshape(NCHT, 1, K)
    return kern(gidx, t3, v.reshape(N_REL * N_NODES, DIM))


def _sum_body(p_ref, o_ref):
    o_ref[...] = p_ref[0] + p_ref[1]


def _sum_partials(partial, n_tile=1000):
    nt = N_NODES // n_tile
    return pl.pallas_call(
        _sum_body,
        grid=(nt,),
        in_specs=[pl.BlockSpec((NC, n_tile, DIM), lambda n: (0, n, 0))],
        out_specs=pl.BlockSpec((n_tile, DIM), lambda n: (n, 0)),
        out_shape=jax.ShapeDtypeStruct((N_NODES, DIM), jnp.float32),
    )(partial)


def kernel(x, source, target, edge_type, base_weights, bases):
    v, gidx = _compute_v_gidx(x.astype(jnp.bfloat16), base_weights, bases,
                              source, edge_type)
    partial = _sc_edge(gidx.reshape(N_EDGES), target, v)
    return _sum_partials(partial)


# speedup vs baseline: 1.0505x; 1.0505x over previous
import functools

import jax
import jax.numpy as jnp
from jax import lax
from jax.experimental import pallas as pl
from jax.experimental.pallas import tpu as pltpu
from jax.experimental.pallas import tpu_sc as plsc

N_NODES = 10000
N_EDGES = 320000
N_REL = 16
N_BASES = 4
DIM = 128

NC = 2
NS = 16
NW = NC * NS
K = 128
NCHT = N_EDGES // K
NCH = NCHT // NW
NCH_REM = NCHT % NW
ROWS_PER_TILE = 624


def _v_body(bw_ref, bases_ref, x_ref, src_ref, et_ref, v_ref, gidx_ref):
    r = pl.program_id(0)
    rows = lax.broadcasted_iota(jnp.int32, (N_REL, N_BASES), 0)
    bvec = jnp.sum(jnp.where(rows == r, bw_ref[...], 0.0), axis=0)
    w = bvec[0] * bases_ref[0]
    for b in range(1, N_BASES):
        w = w + bvec[b] * bases_ref[b]
    v_ref[0] = jnp.dot(x_ref[...], w, preferred_element_type=jnp.float32)

    @pl.when(r == 0)
    def _():
        gidx_ref[...] = et_ref[...] * N_NODES + src_ref[...]


def _compute_v_gidx(x, base_weights, bases, source, edge_type):
    src2 = source.reshape(N_EDGES // 128, 128)
    et2 = edge_type.reshape(N_EDGES // 128, 128)
    return pl.pallas_call(
        _v_body,
        grid=(N_REL,),
        in_specs=[
            pl.BlockSpec((N_REL, N_BASES), lambda r: (0, 0)),
            pl.BlockSpec((N_BASES, DIM, DIM), lambda r: (0, 0, 0)),
            pl.BlockSpec((N_NODES, DIM), lambda r: (0, 0)),
            pl.BlockSpec((N_EDGES // 128, 128), lambda r: (0, 0)),
            pl.BlockSpec((N_EDGES // 128, 128), lambda r: (0, 0)),
        ],
        out_specs=[
            pl.BlockSpec((1, N_NODES, DIM), lambda r: (r, 0, 0)),
            pl.BlockSpec((N_EDGES // 128, 128), lambda r: (0, 0)),
        ],
        out_shape=[
            jax.ShapeDtypeStruct((N_REL, N_NODES, DIM), jnp.float32),
            jax.ShapeDtypeStruct((N_EDGES // 128, 128), jnp.int32),
        ],
    )(base_weights, bases, x, src2, et2)


def _sc_edge_body(gidx_hbm, tgt_hbm, v_hbm, out_hbm,
                  gidx_v, tslot0, tslot1, rows0, rows1,
                  acc, sem0, sem1, semt0, semt1, semi):
    c = lax.axis_index("c")
    s = lax.axis_index("s")
    wid = c * NS + s
    n = NCH + jnp.where(wid < NCH_REM, 1, 0)

    def cid(i):
        return jnp.where(i < NCH, wid * NCH + i, NW * NCH + wid)

    idx_cp = pltpu.async_copy(gidx_hbm.at[pl.ds(wid * NCH * K, NCH * K)],
                              gidx_v.at[pl.ds(0, NCH * K)], semi)
    t_cp0 = pltpu.async_copy(tgt_hbm.at[wid * NCH + 0], tslot0, semt0)
    t_cp1 = pltpu.async_copy(tgt_hbm.at[wid * NCH + 1], tslot1, semt1)

    @pl.when(wid < NCH_REM)
    def _():
        pltpu.async_copy(gidx_hbm.at[pl.ds((NW * NCH + wid) * K, K)],
                         gidx_v.at[pl.ds(NCH * K, K)], semi)

    zero = jnp.zeros((16,), jnp.float32)
    for i in range(K):
        for j in range(DIM // 16):
            rows0[i, pl.ds(j * 16, 16)] = zero

    row0 = s * ROWS_PER_TILE
    for blk in range(4):
        pltpu.sync_copy(rows0, acc.at[pl.ds(row0 + blk * K, K), :])
    pltpu.sync_copy(rows0.at[pl.ds(0, ROWS_PER_TILE - 4 * K), :],
                    acc.at[pl.ds(row0 + 4 * K, ROWS_PER_TILE - 4 * K), :])

    @pl.when(s == 0)
    def _():
        pltpu.sync_copy(rows0.at[pl.ds(0, N_NODES - NS * ROWS_PER_TILE), :],
                        acc.at[pl.ds(NS * ROWS_PER_TILE,
                                     N_NODES - NS * ROWS_PER_TILE), :])

    idx_cp.wait()

    @pl.when(wid < NCH_REM)
    def _():
        pltpu.make_async_copy(gidx_hbm.at[pl.ds(0, K)],
                              gidx_v.at[pl.ds(NCH * K, K)], semi).wait()

    plsc.subcore_barrier()

    def fire_gather(i, rows, sem):
        pltpu.async_copy(v_hbm.at[gidx_v.at[pl.ds(i * K, K)]], rows, sem)

    fire_gather(0, rows0, sem0)
    fire_gather(1, rows1, sem1)

    def step(i, rows, tslot, sem, semt):
        pltpu.make_async_copy(v_hbm.at[gidx_v.at[pl.ds(0, K)]], rows, sem).wait()
        pltpu.make_async_copy(tgt_hbm.at[0], tslot, semt).wait()
        pltpu.sync_copy(rows, acc.at[tslot.at[0]], add=True)

        @pl.when(i + 2 < n)
        def _():
            pltpu.async_copy(tgt_hbm.at[cid(i + 2)], tslot, semt)
            fire_gather(i + 2, rows, sem)

    def pair(j, _):
        step(2 * j, rows0, tslot0, sem0, semt0)
        step(2 * j + 1, rows1, tslot1, sem1, semt1)
        return 0

    lax.fori_loop(0, NCH // 2, pair, 0)

    @pl.when(n > NCH)
    def _():
        pltpu.make_async_copy(v_hbm.at[gidx_v.at[pl.ds(0, K)]], rows0, sem0).wait()
        pltpu.make_async_copy(tgt_hbm.at[0], tslot0, semt0).wait()
        pltpu.sync_copy(rows0, acc.at[tslot0.at[0]], add=True)

    plsc.subcore_barrier()

    pltpu.sync_copy(acc.at[pl.ds(row0, ROWS_PER_TILE), :],
                    out_hbm.at[c, pl.ds(row0, ROWS_PER_TILE), :])

    @pl.when(s == 0)
    def _():
        pltpu.sync_copy(acc.at[pl.ds(NS * ROWS_PER_TILE, N_NODES - NS * ROWS_PER_TILE), :],
                        out_hbm.at[c, pl.ds(NS * ROWS_PER_TILE,
                                            N_NODES - NS * ROWS_PER_TILE), :])


def _sc_edge(gidx, target, v):
    mesh = plsc.VectorSubcoreMesh(core_axis_name="c", subcore_axis_name="s")
    kern = pl.kernel(
        _sc_edge_body,
        out_type=jax.ShapeDtypeStruct((NC, N_NODES, DIM), jnp.float32),
        mesh=mesh,
        scratch_types=[
            pltpu.VMEM(((NCH + 1) * K,), jnp.int32),
            pltpu.VMEM((1, K), jnp.int32),
            pltpu.VMEM((1, K), jnp.int32),
            pltpu.VMEM((K, DIM), jnp.float32),
            pltpu.VMEM((K, DIM), jnp.float32),
            pltpu.VMEM_SHARED((N_NODES, DIM), jnp.float32),
            pltpu.SemaphoreType.DMA,
            pltpu.SemaphoreType.DMA,
            pltpu.SemaphoreType.DMA,
            pltpu.SemaphoreType.DMA,
            pltpu.SemaphoreType.DMA,
        ],
    )
    t3 = target.reshape(NCHT, 1, K)
    return kern(gidx, t3, v.reshape(N_REL * N_NODES, DIM))


def _sum_body(p_ref, o_ref):
    o_ref[...] = p_ref[0] + p_ref[1]


def _sum_partials(partial, n_tile=1000):
    nt = N_NODES // n_tile
    return pl.pallas_call(
        _sum_body,
        grid=(nt,),
        in_specs=[pl.BlockSpec((NC, n_tile, DIM), lambda n: (0, n, 0))],
        out_specs=pl.BlockSpec((n_tile, DIM), lambda n: (n, 0)),
        out_shape=jax.ShapeDtypeStruct((N_NODES, DIM), jnp.float32),
    )(partial)


def kernel(x, source, target, edge_type, base_weights, bases):
    v, gidx = _compute_v_gidx(x, base_weights, bases, source, edge_type)
    partial = _sc_edge(gidx.reshape(N_EDGES), target, v)
    return _sum_partials(partial)
